# Initial kernel scaffold; baseline (speedup 1.0000x reference)
import functools

import jax
import jax.numpy as jnp
from jax import lax
from jax.experimental import pallas as pl
from jax.experimental.pallas import tpu as pltpu
from jax.experimental.pallas import tpu_sc as plsc

EMBED_DIM = 32
LANES = 16
NUM_CORES = 2
NUM_SUBCORES = 16
NUM_WORKERS = NUM_CORES * NUM_SUBCORES
CHUNK = 128


def _make_lookup(batch_flat: int):
    assert batch_flat % (NUM_WORKERS * CHUNK) == 0
    chunks_per_w = batch_flat // (NUM_WORKERS * CHUNK)
    mesh = plsc.VectorSubcoreMesh(core_axis_name="c", subcore_axis_name="s")

    @functools.partial(
        pl.kernel,
        out_type=jax.ShapeDtypeStruct((batch_flat, EMBED_DIM), jnp.float32),
        mesh=mesh,
        scratch_types=[
            pltpu.VMEM((chunks_per_w, CHUNK), jnp.int32),
            pltpu.VMEM((CHUNK, EMBED_DIM), jnp.float32),
            pltpu.SemaphoreType.DMA,
        ],
    )
    def lookup(table_hbm, idx_hbm, out_hbm, idx_v, rows_v, sem):
        wid = lax.axis_index("s") * NUM_CORES + lax.axis_index("c")
        base_chunk = wid * chunks_per_w
        pltpu.sync_copy(idx_hbm.at[pl.ds(base_chunk, chunks_per_w), :], idx_v)

        def chunk_body(j, carry):
            pltpu.async_copy(table_hbm.at[idx_v.at[j]], rows_v, sem).wait()

            def abs_row(r, c):
                rows_v[r, pl.ds(0, LANES)] = jnp.abs(rows_v[r, pl.ds(0, LANES)])
                rows_v[r, pl.ds(LANES, LANES)] = jnp.abs(
                    rows_v[r, pl.ds(LANES, LANES)]
                )
                return c

            lax.fori_loop(0, CHUNK, abs_row, 0)
            pltpu.sync_copy(
                rows_v, out_hbm.at[pl.ds((base_chunk + j) * CHUNK, CHUNK), :]
            )
            return carry

        lax.fori_loop(0, chunks_per_w, chunk_body, 0)

    return lookup


def kernel(inputs, table):
    batch, fields = inputs.shape
    b_flat = batch * fields
    idx2d = inputs.reshape(b_flat // CHUNK, CHUNK).astype(jnp.int32)
    out = _make_lookup(b_flat)(table, idx2d)
    return out.reshape(batch, fields, EMBED_DIM)


# speedup vs baseline: 1.2081x; 1.2081x over previous
import functools

import jax
import jax.numpy as jnp
from jax import lax
from jax.experimental import pallas as pl
from jax.experimental.pallas import tpu as pltpu
from jax.experimental.pallas import tpu_sc as plsc

EMBED_DIM = 32
LANES = 16
NUM_CORES = 2
NUM_SUBCORES = 16
NUM_WORKERS = NUM_CORES * NUM_SUBCORES
CHUNK = 128


def _make_lookup(batch_flat: int):
    assert batch_flat % (NUM_WORKERS * CHUNK) == 0
    chunks_per_w = batch_flat // (NUM_WORKERS * CHUNK)
    mesh = plsc.VectorSubcoreMesh(core_axis_name="c", subcore_axis_name="s")

    @functools.partial(
        pl.kernel,
        out_type=jax.ShapeDtypeStruct((batch_flat, EMBED_DIM), jnp.float32),
        mesh=mesh,
        scratch_types=[
            pltpu.VMEM((chunks_per_w, CHUNK), jnp.int32),
            pltpu.VMEM((CHUNK, EMBED_DIM), jnp.float32),
            pltpu.SemaphoreType.DMA,
        ],
        compiler_params=pltpu.CompilerParams(use_tc_tiling_on_sc=False),
    )
    def lookup(table_hbm, idx_hbm, out_hbm, idx_v, rows_v, sem):
        wid = lax.axis_index("s") * NUM_CORES + lax.axis_index("c")
        base_chunk = wid * chunks_per_w
        pltpu.sync_copy(idx_hbm.at[pl.ds(base_chunk, chunks_per_w), :], idx_v)

        def chunk_body(j, carry):
            pltpu.async_copy(table_hbm.at[idx_v.at[j]], rows_v, sem).wait()

            def abs_row(r, c):
                rows_v[r, pl.ds(0, LANES)] = jnp.abs(rows_v[r, pl.ds(0, LANES)])
                rows_v[r, pl.ds(LANES, LANES)] = jnp.abs(
                    rows_v[r, pl.ds(LANES, LANES)]
                )
                return c

            lax.fori_loop(0, CHUNK, abs_row, 0)
            pltpu.sync_copy(
                rows_v, out_hbm.at[pl.ds((base_chunk + j) * CHUNK, CHUNK), :]
            )
            return carry

        lax.fori_loop(0, chunks_per_w, chunk_body, 0)

    return lookup


def kernel(inputs, table):
    batch, fields = inputs.shape
    b_flat = batch * fields
    idx2d = inputs.reshape(b_flat // CHUNK, CHUNK).astype(jnp.int32)
    out = _make_lookup(b_flat)(table, idx2d)
    return out.reshape(batch, fields, EMBED_DIM)


# speedup vs baseline: 1.3626x; 1.1278x over previous
import functools

import jax
import jax.numpy as jnp
from jax import lax
from jax.experimental import pallas as pl
from jax.experimental.pallas import tpu as pltpu
from jax.experimental.pallas import tpu_sc as plsc

EMBED_DIM = 32
LANES = 16
NUM_CORES = 2
NUM_SUBCORES = 16
NUM_WORKERS = NUM_CORES * NUM_SUBCORES
CHUNK = 128


def _make_lookup(batch_flat: int):
    assert batch_flat % (NUM_WORKERS * CHUNK * 2) == 0
    chunks_per_w = batch_flat // (NUM_WORKERS * CHUNK)
    n_half = chunks_per_w // 2
    mesh = plsc.VectorSubcoreMesh(core_axis_name="c", subcore_axis_name="s")

    @functools.partial(
        pl.kernel,
        out_type=jax.ShapeDtypeStruct((batch_flat, EMBED_DIM), jnp.float32),
        mesh=mesh,
        scratch_types=[
            pltpu.VMEM((chunks_per_w, CHUNK), jnp.int32),
            pltpu.VMEM((CHUNK, EMBED_DIM), jnp.float32),
            pltpu.VMEM((CHUNK, EMBED_DIM), jnp.float32),
            pltpu.VMEM((CHUNK, EMBED_DIM), jnp.float32),
            pltpu.VMEM((CHUNK, EMBED_DIM), jnp.float32),
            pltpu.SemaphoreType.DMA,
            pltpu.SemaphoreType.DMA,
            pltpu.SemaphoreType.DMA,
            pltpu.SemaphoreType.DMA,
        ],
        compiler_params=pltpu.CompilerParams(use_tc_tiling_on_sc=False),
    )
    def lookup(
        table_hbm, idx_hbm, out_hbm,
        idx_v, in0, in1, st0, st1, sg0, sg1, ss0, ss1,
    ):
        ins, sts = (in0, in1), (st0, st1)
        sgs, sss = (sg0, sg1), (ss0, ss1)
        wid = lax.axis_index("s") * NUM_CORES + lax.axis_index("c")
        base_chunk = wid * chunks_per_w
        pltpu.sync_copy(idx_hbm.at[pl.ds(base_chunk, chunks_per_w), :], idx_v)

        pltpu.async_copy(table_hbm.at[idx_v.at[0]], in0, sg0)
        pltpu.async_copy(table_hbm.at[idx_v.at[1]], in1, sg1)

        def iter_body(i, carry):
            for b in range(2):
                j = 2 * i + b
                pltpu.make_async_copy(
                    table_hbm.at[pl.ds(0, CHUNK), :], ins[b], sgs[b]
                ).wait()
                @pl.when(i > 0)
                def _():
                    pltpu.make_async_copy(
                        sts[b], out_hbm.at[pl.ds(0, CHUNK), :], sss[b]
                    ).wait()

                @plsc.parallel_loop(0, CHUNK, step=1, unroll=8)
                def _(r):
                    sts[b][r, pl.ds(0, LANES)] = jnp.abs(
                        ins[b][r, pl.ds(0, LANES)]
                    )
                    sts[b][r, pl.ds(LANES, LANES)] = jnp.abs(
                        ins[b][r, pl.ds(LANES, LANES)]
                    )

                @pl.when(i < n_half - 1)
                def _():
                    pltpu.async_copy(table_hbm.at[idx_v.at[j + 2]], ins[b], sgs[b])

                pltpu.async_copy(
                    sts[b],
                    out_hbm.at[pl.ds((base_chunk + j) * CHUNK, CHUNK), :],
                    sss[b],
                )
            return carry

        lax.fori_loop(0, n_half, iter_body, 0)
        for b in range(2):
            pltpu.make_async_copy(
                sts[b], out_hbm.at[pl.ds(0, CHUNK), :], sss[b]
            ).wait()

    return lookup


def kernel(inputs, table):
    batch, fields = inputs.shape
    b_flat = batch * fields
    idx2d = inputs.reshape(b_flat // CHUNK, CHUNK).astype(jnp.int32)
    out = _make_lookup(b_flat)(table, idx2d)
    return out.reshape(batch, fields, EMBED_DIM)


# speedup vs baseline: 1.6297x; 1.1960x over previous
import functools

import jax
import jax.numpy as jnp
from jax import lax
from jax.experimental import pallas as pl
from jax.experimental.pallas import tpu as pltpu
from jax.experimental.pallas import tpu_sc as plsc

EMBED_DIM = 32
LANES = 16
NUM_CORES = 2
NUM_SUBCORES = 16
NUM_WORKERS = NUM_CORES * NUM_SUBCORES
CHUNK = 128
SUB = EMBED_DIM // 8


def _make_lookup(fields: int, n_tiles: int):
    n_chunks = fields * n_tiles
    assert n_chunks % (NUM_WORKERS * 2) == 0
    chunks_per_w = n_chunks // NUM_WORKERS
    n_half = chunks_per_w // 2
    mesh = plsc.VectorSubcoreMesh(core_axis_name="c", subcore_axis_name="s")

    @functools.partial(
        pl.kernel,
        out_type=jax.ShapeDtypeStruct(
            (fields, SUB, n_tiles, 8, CHUNK), jnp.float32
        ),
        mesh=mesh,
        scratch_types=[
            pltpu.VMEM((chunks_per_w, CHUNK), jnp.int32),
            pltpu.VMEM((CHUNK, EMBED_DIM), jnp.float32),
            pltpu.VMEM((CHUNK, EMBED_DIM), jnp.float32),
            pltpu.VMEM((SUB, 8, CHUNK), jnp.float32),
            pltpu.VMEM((SUB, 8, CHUNK), jnp.float32),
            pltpu.SemaphoreType.DMA,
            pltpu.SemaphoreType.DMA,
            pltpu.SemaphoreType.DMA,
            pltpu.SemaphoreType.DMA,
        ],
        compiler_params=pltpu.CompilerParams(
            use_tc_tiling_on_sc=False, needs_layout_passes=False
        ),
    )
    def lookup(
        table_hbm, idx_hbm, out_hbm,
        idx_v, in0, in1, tr0, tr1, sg0, sg1, ss0, ss1,
    ):
        ins, trs = (in0, in1), (tr0, tr1)
        sgs, sss = (sg0, sg1), (ss0, ss1)
        wid = lax.axis_index("s") * NUM_CORES + lax.axis_index("c")
        base = wid * chunks_per_w
        pltpu.sync_copy(idx_hbm.at[pl.ds(base, chunks_per_w), :], idx_v)

        pltpu.async_copy(table_hbm.at[idx_v.at[0]], in0, sg0)
        pltpu.async_copy(table_hbm.at[idx_v.at[1]], in1, sg1)

        def iter_body(i, carry):
            for b in range(2):
                j = 2 * i + b
                ch = base + j
                f = ch // n_tiles
                t = lax.rem(ch, n_tiles)
                pltpu.make_async_copy(
                    table_hbm.at[pl.ds(0, CHUNK), :], ins[b], sgs[b]
                ).wait()
                @pl.when(i > 0)
                def _():
                    pltpu.make_async_copy(
                        trs[b], out_hbm.at[0, :, 0, :, :], sss[b]
                    ).wait()

                @plsc.parallel_loop(0, EMBED_DIM, unroll=4)
                def _(jcol):
                    a = jcol // 8
                    s = lax.rem(jcol, 8)
                    cols = jnp.full((LANES,), jcol, jnp.int32)
                    for cs in range(0, CHUNK, LANES):
                        rows = cs + lax.iota(jnp.int32, LANES)
                        v = plsc.load_gather(ins[b], [rows, cols])
                        trs[b][a, s, pl.ds(cs, LANES)] = jnp.abs(v)

                @pl.when(i < n_half - 1)
                def _():
                    pltpu.async_copy(table_hbm.at[idx_v.at[j + 2]], ins[b], sgs[b])

                pltpu.async_copy(trs[b], out_hbm.at[f, :, t, :, :], sss[b])
            return carry

        lax.fori_loop(0, n_half, iter_body, 0)
        for b in range(2):
            pltpu.make_async_copy(
                trs[b], out_hbm.at[0, :, 0, :, :], sss[b]
            ).wait()

    return lookup


def kernel(inputs, table):
    batch, fields = inputs.shape
    n_tiles = batch // CHUNK
    idx2d = inputs.T.reshape(fields * n_tiles, CHUNK).astype(jnp.int32)
    out5 = _make_lookup(fields, n_tiles)(table, idx2d)
    return out5.transpose(2, 4, 0, 1, 3).reshape(batch, fields, EMBED_DIM)


# speedup vs baseline: 1.8608x; 1.1418x over previous
import functools

import jax
import jax.numpy as jnp
from jax import lax
from jax.experimental import pallas as pl
from jax.experimental.pallas import tpu as pltpu
from jax.experimental.pallas import tpu_sc as plsc

EMBED_DIM = 32
LANES = 16
NUM_CORES = 2
NUM_SUBCORES = 16
NUM_WORKERS = NUM_CORES * NUM_SUBCORES
CHUNK = 128
SUB = EMBED_DIM // 8


def _make_relayout(vocab: int):
    n_full = vocab // CHUNK
    rem = vocab - n_full * CHUNK
    per_w = n_full // NUM_WORKERS
    per_w -= per_w % 2
    tiles_main = per_w * NUM_WORKERS
    tail_full = n_full - tiles_main
    n_half = per_w // 2
    tile_elems = CHUNK * EMBED_DIM
    mesh = plsc.VectorSubcoreMesh(core_axis_name="c", subcore_axis_name="s")

    @functools.partial(
        pl.kernel,
        out_type=jax.ShapeDtypeStruct((vocab * EMBED_DIM,), jnp.float32),
        mesh=mesh,
        scratch_types=[
            pltpu.VMEM((EMBED_DIM, CHUNK), jnp.float32),
            pltpu.VMEM((EMBED_DIM, CHUNK), jnp.float32),
            pltpu.VMEM((tile_elems,), jnp.float32),
            pltpu.VMEM((tile_elems,), jnp.float32),
            pltpu.SemaphoreType.DMA,
            pltpu.SemaphoreType.DMA,
            pltpu.SemaphoreType.DMA,
            pltpu.SemaphoreType.DMA,
        ],
        compiler_params=pltpu.CompilerParams(
            use_tc_tiling_on_sc=True, needs_layout_passes=False
        ),
    )
    def relayout(
        tab_t_hbm, tail_hbm, dense_hbm, t0v, t1v, d0v, d1v, sg0, sg1, ss0, ss1
    ):
        tvs, dvs = (t0v, t1v), (d0v, d1v)
        sgs, sss = (sg0, sg1), (ss0, ss1)
        wid = lax.axis_index("s") * NUM_CORES + lax.axis_index("c")
        base = wid * per_w

        def transpose_tile(src, dst, width):
            @plsc.parallel_loop(0, width, unroll=4)
            def _(cc):
                cols = jnp.full((LANES,), cc, jnp.int32)
                for j0 in range(0, EMBED_DIM, LANES):
                    rows = j0 + lax.iota(jnp.int32, LANES)
                    dst[pl.ds(cc * EMBED_DIM + j0, LANES)] = plsc.load_gather(
                        src, [rows, cols]
                    )

        pltpu.async_copy(
            tab_t_hbm.at[:, pl.ds(base * CHUNK, CHUNK)], t0v, sg0
        )
        pltpu.async_copy(
            tab_t_hbm.at[:, pl.ds((base + 1) * CHUNK, CHUNK)], t1v, sg1
        )

        def iter_body(i, carry):
            for b in range(2):
                t = base + 2 * i + b
                pltpu.make_async_copy(
                    tab_t_hbm.at[:, pl.ds(0, CHUNK)], tvs[b], sgs[b]
                ).wait()
                @pl.when(i > 0)
                def _():
                    pltpu.make_async_copy(
                        dvs[b], dense_hbm.at[pl.ds(0, tile_elems)], sss[b]
                    ).wait()

                transpose_tile(tvs[b], dvs[b], CHUNK)

                @pl.when(i < n_half - 1)
                def _():
                    pltpu.async_copy(
                        tab_t_hbm.at[:, pl.ds((t + 2) * CHUNK, CHUNK)],
                        tvs[b],
                        sgs[b],
                    )

                pltpu.async_copy(
                    dvs[b], dense_hbm.at[pl.ds(t * tile_elems, tile_elems)], sss[b]
                )
            return carry

        lax.fori_loop(0, n_half, iter_body, 0)
        for b in range(2):
            pltpu.make_async_copy(
                dvs[b], dense_hbm.at[pl.ds(0, tile_elems)], sss[b]
            ).wait()

        @pl.when(wid < tail_full)
        def _():
            t = tiles_main + wid
            pltpu.sync_copy(tab_t_hbm.at[:, pl.ds(t * CHUNK, CHUNK)], t0v)
            transpose_tile(t0v, d0v, CHUNK)
            pltpu.sync_copy(
                d0v, dense_hbm.at[pl.ds(t * tile_elems, tile_elems)]
            )

        if rem:
            @pl.when(wid == tail_full)
            def _():
                start = n_full * CHUNK * EMBED_DIM
                n = rem * EMBED_DIM
                pltpu.sync_copy(tail_hbm, d1v.at[pl.ds(0, n)])
                pltpu.sync_copy(
                    d1v.at[pl.ds(0, n)], dense_hbm.at[pl.ds(start, n)]
                )

    return relayout


def _make_lookup(fields: int, n_tiles: int):
    n_chunks = fields * n_tiles
    assert n_chunks % (NUM_WORKERS * 2) == 0
    chunks_per_w = n_chunks // NUM_WORKERS
    n_half = chunks_per_w // 2
    mesh = plsc.VectorSubcoreMesh(core_axis_name="c", subcore_axis_name="s")

    @functools.partial(
        pl.kernel,
        out_type=jax.ShapeDtypeStruct(
            (fields, SUB, n_tiles, 8, CHUNK), jnp.float32
        ),
        mesh=mesh,
        scratch_types=[
            pltpu.VMEM((chunks_per_w, CHUNK), jnp.int32),
            pltpu.VMEM((CHUNK, EMBED_DIM), jnp.float32),
            pltpu.VMEM((CHUNK, EMBED_DIM), jnp.float32),
            pltpu.VMEM((SUB, 8, CHUNK), jnp.float32),
            pltpu.VMEM((SUB, 8, CHUNK), jnp.float32),
            pltpu.SemaphoreType.DMA,
            pltpu.SemaphoreType.DMA,
            pltpu.SemaphoreType.DMA,
            pltpu.SemaphoreType.DMA,
        ],
        compiler_params=pltpu.CompilerParams(
            use_tc_tiling_on_sc=False, needs_layout_passes=False
        ),
    )
    def lookup(
        table_hbm, idx_hbm, out_hbm,
        idx_v, in0, in1, tr0, tr1, sg0, sg1, ss0, ss1,
    ):
        ins, trs = (in0, in1), (tr0, tr1)
        sgs, sss = (sg0, sg1), (ss0, ss1)
        wid = lax.axis_index("s") * NUM_CORES + lax.axis_index("c")
        base = wid * chunks_per_w
        pltpu.sync_copy(idx_hbm.at[pl.ds(base, chunks_per_w), :], idx_v)

        pltpu.async_copy(table_hbm.at[idx_v.at[0]], in0, sg0)
        pltpu.async_copy(table_hbm.at[idx_v.at[1]], in1, sg1)

        def iter_body(i, carry):
            for b in range(2):
                j = 2 * i + b
                ch = base + j
                f = ch // n_tiles
                t = lax.rem(ch, n_tiles)
                pltpu.make_async_copy(
                    table_hbm.at[pl.ds(0, CHUNK), :], ins[b], sgs[b]
                ).wait()
                @pl.when(i > 0)
                def _():
                    pltpu.make_async_copy(
                        trs[b], out_hbm.at[0, :, 0, :, :], sss[b]
                    ).wait()

                @plsc.parallel_loop(0, EMBED_DIM, unroll=4)
                def _(jcol):
                    a = jcol // 8
                    s = lax.rem(jcol, 8)
                    cols = jnp.full((LANES,), jcol, jnp.int32)
                    for cs in range(0, CHUNK, LANES):
                        rows = cs + lax.iota(jnp.int32, LANES)
                        v = plsc.load_gather(ins[b], [rows, cols])
                        trs[b][a, s, pl.ds(cs, LANES)] = jnp.abs(v)

                @pl.when(i < n_half - 1)
                def _():
                    pltpu.async_copy(table_hbm.at[idx_v.at[j + 2]], ins[b], sgs[b])

                pltpu.async_copy(trs[b], out_hbm.at[f, :, t, :, :], sss[b])
            return carry

        lax.fori_loop(0, n_half, iter_body, 0)
        for b in range(2):
            pltpu.make_async_copy(
                trs[b], out_hbm.at[0, :, 0, :, :], sss[b]
            ).wait()

    return lookup


def kernel(inputs, table):
    batch, fields = inputs.shape
    vocab = table.shape[0]
    n_tiles = batch // CHUNK
    rem = vocab % CHUNK
    tail1d = table[vocab - rem :].reshape(-1) if rem else jnp.zeros(
        (EMBED_DIM,), jnp.float32
    )
    dense1d = _make_relayout(vocab)(table.T, tail1d)
    dense2d = dense1d.reshape(vocab, EMBED_DIM)
    idx2d = inputs.T.reshape(fields * n_tiles, CHUNK).astype(jnp.int32)
    out5 = _make_lookup(fields, n_tiles)(dense2d, idx2d)
    return out5.transpose(2, 4, 0, 1, 3).reshape(batch, fields, EMBED_DIM)
